# baseline (speedup 1.0000x reference)
import jax
import jax.numpy as jnp
from jax.experimental import pallas as pl

N_TOK = 4096
D_IN = 2048
D_LAT = 16384
K = 64

ENC_BT = 256
ENC_BL = 2048
DEC_BT = 512
DEC_BL = 1024


def _enc_body(x_ref, w_ref, b_ref, o_ref):
    acc = jax.lax.dot_general(
        x_ref[...], w_ref[...],
        dimension_numbers=(((1,), (1,)), ((), ())),
        preferred_element_type=jnp.float32,
    )
    o_ref[...] = jnp.maximum(acc + b_ref[...], 0.0)


def _encoder(x, W_enc, b_enc):
    b2 = b_enc.reshape(1, D_LAT)
    grid = (D_LAT // ENC_BL, N_TOK // ENC_BT)
    return pl.pallas_call(
        _enc_body,
        grid=grid,
        in_specs=[
            pl.BlockSpec((ENC_BT, D_IN), lambda l, t: (t, 0)),
            pl.BlockSpec((ENC_BL, D_IN), lambda l, t: (l, 0)),
            pl.BlockSpec((1, ENC_BL), lambda l, t: (0, l)),
        ],
        out_specs=pl.BlockSpec((ENC_BT, ENC_BL), lambda l, t: (t, l)),
        out_shape=jax.ShapeDtypeStruct((N_TOK, D_LAT), jnp.float32),
    )(x.astype(jnp.bfloat16), W_enc.astype(jnp.bfloat16), b2)


def _dec_body(z_ref, w_ref, b_ref, o_ref):
    l = pl.program_id(1)
    acc = jax.lax.dot_general(
        z_ref[...].astype(jnp.bfloat16), w_ref[...].astype(jnp.bfloat16),
        dimension_numbers=(((1,), (1,)), ((), ())),
        preferred_element_type=jnp.float32,
    )

    @pl.when(l == 0)
    def _():
        o_ref[...] = acc + b_ref[...]

    @pl.when(l != 0)
    def _():
        o_ref[...] += acc


def _decoder(z, W_dec, b_dec):
    b2 = b_dec.reshape(1, D_IN)
    grid = (N_TOK // DEC_BT, D_LAT // DEC_BL)
    return pl.pallas_call(
        _dec_body,
        grid=grid,
        in_specs=[
            pl.BlockSpec((DEC_BT, DEC_BL), lambda t, l: (t, l)),
            pl.BlockSpec((D_IN, DEC_BL), lambda t, l: (0, l)),
            pl.BlockSpec((1, D_IN), lambda t, l: (0, 0)),
        ],
        out_specs=pl.BlockSpec((DEC_BT, D_IN), lambda t, l: (t, 0)),
        out_shape=jax.ShapeDtypeStruct((N_TOK, D_IN), jnp.float32),
    )(z, W_dec, b2)


def kernel(x, W_enc, b_enc, W_dec, b_dec):
    pre = _encoder(x, W_enc, b_enc)
    vals, idx = jax.lax.top_k(pre, K)
    row = jnp.arange(N_TOK)[:, None]
    z = jnp.zeros_like(pre).at[row, idx].set(vals)
    x_hat = _decoder(z, W_dec, b_dec)
    return (x_hat, z)


# speedup vs baseline: 6.5658x; 6.5658x over previous
import dataclasses
import functools

import jax
import jax.numpy as jnp
from jax import lax
from jax.experimental import pallas as pl
from jax.experimental.pallas import tpu as pltpu
from jax.experimental.pallas import tpu_sc as plsc

N_TOK = 4096
D_IN = 2048
D_LAT = 16384
K = 64

ENC_BT = 256
ENC_BL = 2048
DEC_BT = 512
DEC_BL = 1024
CHUNK = 128
NCHUNK = D_LAT // CHUNK

NW = 32
ROWS_PER_W = N_TOK // NW
CAND_N = 2048
INF_BITS = 0x7F800000


def _enc_body(x_ref, w_ref, b_ref, o_ref, m_ref):
    acc = jax.lax.dot_general(
        x_ref[...], w_ref[...],
        dimension_numbers=(((1,), (1,)), ((), ())),
        preferred_element_type=jnp.float32,
    )
    pre = jnp.maximum(acc + b_ref[...], 0.0)
    o_ref[...] = pre
    m = jnp.max(pre.reshape(ENC_BT, ENC_BL // CHUNK, CHUNK), axis=2)
    m_ref[...] = m.reshape(1, ENC_BT, ENC_BL // CHUNK)


def _encoder(x, W_enc, b_enc):
    b2 = b_enc.reshape(1, D_LAT)
    grid = (D_LAT // ENC_BL, N_TOK // ENC_BT)
    return pl.pallas_call(
        _enc_body,
        grid=grid,
        in_specs=[
            pl.BlockSpec((ENC_BT, D_IN), lambda l, t: (t, 0)),
            pl.BlockSpec((ENC_BL, D_IN), lambda l, t: (l, 0)),
            pl.BlockSpec((1, ENC_BL), lambda l, t: (0, l)),
        ],
        out_specs=[
            pl.BlockSpec((ENC_BT, ENC_BL), lambda l, t: (t, l)),
            pl.BlockSpec((1, ENC_BT, ENC_BL // CHUNK), lambda l, t: (l, t, 0)),
        ],
        out_shape=[
            jax.ShapeDtypeStruct((N_TOK, D_LAT), jnp.float32),
            jax.ShapeDtypeStruct((D_LAT // ENC_BL, N_TOK, ENC_BL // CHUNK),
                                 jnp.float32),
        ],
    )(x.astype(jnp.bfloat16), W_enc.astype(jnp.bfloat16), b2)


def _count_ge_f32(ref, nv, mid):
    midv = jnp.full((16,), mid, jnp.int32)

    def body(j, acc):
        bits = plsc.bitcast(ref[pl.ds(j * 16, 16)], jnp.int32)
        return acc + jnp.sum((bits >= midv).astype(jnp.int32))

    return lax.fori_loop(0, nv, body, 0)


def _count_ge_i32(ref, nv, mid):
    midv = jnp.full((16,), mid, jnp.int32)

    def body(j, acc):
        bits = ref[pl.ds(j * 16, 16)]
        return acc + jnp.sum((bits >= midv).astype(jnp.int32))

    return lax.fori_loop(0, nv, body, 0)


def _bisect(count_fn, need, lo, hi):
    def it(_, c):
        lo, hi = c
        mid = lo + (hi - lo) // 2
        good = count_fn(mid) >= need
        return (jnp.where(good, mid, lo), jnp.where(good, hi, mid))

    lo, hi = lax.fori_loop(0, 32, it, (lo, hi))
    return lo


def _sc_thresholds(pre, M):
    mesh = plsc.VectorSubcoreMesh(core_axis_name="c", subcore_axis_name="s")
    cp = pltpu.CompilerParams()
    if "needs_layout_passes" in pltpu.CompilerParams.__dataclass_fields__:
        cp = dataclasses.replace(cp, needs_layout_passes=False)

    @functools.partial(
        pl.kernel,
        mesh=mesh,
        compiler_params=cp,
        out_type=jax.ShapeDtypeStruct((N_TOK,), jnp.int32),
        scratch_types=[
            pltpu.VMEM((D_LAT,), jnp.float32),
            pltpu.VMEM((D_LAT,), jnp.float32),
            pltpu.VMEM((ROWS_PER_W, NCHUNK), jnp.float32),
            pltpu.VMEM((CAND_N,), jnp.int32),
            pltpu.VMEM((ROWS_PER_W,), jnp.int32),
            pltpu.SemaphoreType.DMA,
            pltpu.SemaphoreType.DMA,
            pltpu.SemaphoreType.DMA,
        ],
    )
    def k(pre_hbm, m_hbm, thr_hbm, row0, row1, mslab, cand, thrbuf,
          sem0, sem1, semm):
        wid = lax.axis_index("s") * 2 + lax.axis_index("c")
        base = wid * ROWS_PER_W
        pltpu.async_copy(m_hbm.at[pl.ds(base, ROWS_PER_W)], mslab, semm).wait()
        pltpu.async_copy(pre_hbm.at[base], row0, sem0)

        def do_row(r, cur, cur_sem, nxt, nxt_sem):
            pltpu.make_async_copy(pre_hbm.at[base + r], cur, cur_sem).wait()

            @pl.when(r + 1 < ROWS_PER_W)
            def _():
                pltpu.async_copy(pre_hbm.at[base + r + 1], nxt, nxt_sem)

            def count_m(mid):
                midv = jnp.full((16,), mid, jnp.int32)

                def body(j, acc):
                    bits = plsc.bitcast(mslab[r, pl.ds(j * 16, 16)], jnp.int32)
                    return acc + jnp.sum((bits >= midv).astype(jnp.int32))

                return lax.fori_loop(0, NCHUNK // 16, body, 0)

            t0 = _bisect(count_m, K, 1, INF_BITS + 8)

            t0v = jnp.full((16,), t0, jnp.int32)

            def cbody(j, off):
                bits = plsc.bitcast(cur[pl.ds(j * 16, 16)], jnp.int32)
                mask = bits >= t0v
                mi = mask.astype(jnp.int32)
                pos = jnp.minimum(off + jnp.cumsum(mi) - 1, CAND_N - 1)
                plsc.store_scatter(cand, [pos], bits, mask=mask)
                return off + jnp.sum(mi)

            n_cand = lax.fori_loop(0, D_LAT // 16, cbody, 0)
            n_eff = jnp.minimum(n_cand, CAND_N - 16)
            cand[pl.ds(n_eff, 16)] = jnp.zeros((16,), jnp.int32)

            need = jnp.minimum(K, n_cand)
            nv = (n_eff + 15) // 16
            thr = _bisect(
                lambda mid: _count_ge_i32(cand, nv, mid),
                need, t0, INF_BITS + 8)
            thr = jnp.where(n_cand == 0, INF_BITS, thr)
            lane = lax.iota(jnp.int32, 16)
            plsc.store_scatter(thrbuf, [jnp.full((16,), r, jnp.int32)],
                               jnp.full((16,), thr, jnp.int32),
                               mask=lane == 0)

        @pl.loop(0, ROWS_PER_W, step=2)
        def _(r):
            do_row(r, row0, sem0, row1, sem1)
            do_row(r + 1, row1, sem1, row0, sem0)

        pltpu.sync_copy(thrbuf, thr_hbm.at[pl.ds(base, ROWS_PER_W)])

    thr_bits = k(pre, M)
    return lax.bitcast_convert_type(thr_bits, jnp.float32).reshape(N_TOK, 1)


def _dec_body(pre_ref, thr_ref, w_ref, b_ref, z_ref, o_ref):
    l = pl.program_id(1)
    pre = pre_ref[...]
    thr = thr_ref[...]
    z = jnp.where((pre >= thr) & (pre > 0.0), pre, 0.0)
    z_ref[...] = z
    acc = jax.lax.dot_general(
        z.astype(jnp.bfloat16), w_ref[...],
        dimension_numbers=(((1,), (1,)), ((), ())),
        preferred_element_type=jnp.float32,
    )

    @pl.when(l == 0)
    def _():
        o_ref[...] = acc + b_ref[...]

    @pl.when(l != 0)
    def _():
        o_ref[...] += acc


def _decoder(pre, thr, W_dec, b_dec):
    b2 = b_dec.reshape(1, D_IN)
    grid = (N_TOK // DEC_BT, D_LAT // DEC_BL)
    return pl.pallas_call(
        _dec_body,
        grid=grid,
        in_specs=[
            pl.BlockSpec((DEC_BT, DEC_BL), lambda t, l: (t, l)),
            pl.BlockSpec((DEC_BT, 1), lambda t, l: (t, 0)),
            pl.BlockSpec((D_IN, DEC_BL), lambda t, l: (0, l)),
            pl.BlockSpec((1, D_IN), lambda t, l: (0, 0)),
        ],
        out_specs=[
            pl.BlockSpec((DEC_BT, DEC_BL), lambda t, l: (t, l)),
            pl.BlockSpec((DEC_BT, D_IN), lambda t, l: (t, 0)),
        ],
        out_shape=[
            jax.ShapeDtypeStruct((N_TOK, D_LAT), jnp.float32),
            jax.ShapeDtypeStruct((N_TOK, D_IN), jnp.float32),
        ],
    )(pre, thr, W_dec.astype(jnp.bfloat16), b2)


def kernel(x, W_enc, b_enc, W_dec, b_dec):
    pre, M3 = _encoder(x, W_enc, b_enc)
    M = M3.transpose(1, 0, 2).reshape(N_TOK, NCHUNK)
    thr = _sc_thresholds(pre, M)
    z, x_hat = _decoder(pre, thr, W_dec, b_dec)
    return (x_hat, z)


# speedup vs baseline: 7.7886x; 1.1862x over previous
import dataclasses
import functools

import jax
import jax.numpy as jnp
from jax import lax
from jax.experimental import pallas as pl
from jax.experimental.pallas import tpu as pltpu
from jax.experimental.pallas import tpu_sc as plsc

N_TOK = 4096
D_IN = 2048
D_LAT = 16384
K = 64

ENC_BT = 256
ENC_BL = 2048
DEC_BT = 512
DEC_BL = 1024
CHUNK = 128
NCHUNK = D_LAT // CHUNK

NW = 32
ROWS_PER_W = N_TOK // NW
LANE_SEG = 32
CAND_N = 16 * LANE_SEG
INF_BITS = 0x7F800000


def _enc_body(x_ref, w_ref, b_ref, o_ref, m_ref):
    acc = jax.lax.dot_general(
        x_ref[...], w_ref[...],
        dimension_numbers=(((1,), (1,)), ((), ())),
        preferred_element_type=jnp.float32,
    )
    pre = jnp.maximum(acc + b_ref[...], 0.0)
    o_ref[...] = pre
    m = jnp.max(pre.reshape(ENC_BT, ENC_BL // CHUNK, CHUNK), axis=2)
    m_ref[...] = m.reshape(1, ENC_BT, ENC_BL // CHUNK)


def _encoder(x, W_enc, b_enc):
    b2 = b_enc.reshape(1, D_LAT)
    grid = (D_LAT // ENC_BL, N_TOK // ENC_BT)
    return pl.pallas_call(
        _enc_body,
        grid=grid,
        in_specs=[
            pl.BlockSpec((ENC_BT, D_IN), lambda l, t: (t, 0)),
            pl.BlockSpec((ENC_BL, D_IN), lambda l, t: (l, 0)),
            pl.BlockSpec((1, ENC_BL), lambda l, t: (0, l)),
        ],
        out_specs=[
            pl.BlockSpec((ENC_BT, ENC_BL), lambda l, t: (t, l)),
            pl.BlockSpec((1, ENC_BT, ENC_BL // CHUNK), lambda l, t: (l, t, 0)),
        ],
        out_shape=[
            jax.ShapeDtypeStruct((N_TOK, D_LAT), jnp.float32),
            jax.ShapeDtypeStruct((D_LAT // ENC_BL, N_TOK, ENC_BL // CHUNK),
                                 jnp.float32),
        ],
    )(x.astype(jnp.bfloat16), W_enc.astype(jnp.bfloat16), b2)


def _count_ge_f32(ref, nv, mid):
    midv = jnp.full((16,), mid, jnp.int32)

    def body(j, acc):
        bits = plsc.bitcast(ref[pl.ds(j * 16, 16)], jnp.int32)
        return acc + jnp.sum((bits >= midv).astype(jnp.int32))

    return lax.fori_loop(0, nv, body, 0)


def _bisect(count_fn, need, lo, hi):
    def cond(c):
        lo, hi = c
        return hi - lo > 1

    def it(c):
        lo, hi = c
        mid = lo + (hi - lo) // 2
        good = count_fn(mid) >= need
        return (jnp.where(good, mid, lo), jnp.where(good, hi, mid))

    lo, hi = lax.while_loop(cond, it, (lo, hi))
    return lo


def _sc_thresholds(pre, M):
    mesh = plsc.VectorSubcoreMesh(core_axis_name="c", subcore_axis_name="s")
    cp = pltpu.CompilerParams()
    if "needs_layout_passes" in pltpu.CompilerParams.__dataclass_fields__:
        cp = dataclasses.replace(cp, needs_layout_passes=False)

    @functools.partial(
        pl.kernel,
        mesh=mesh,
        compiler_params=cp,
        out_type=jax.ShapeDtypeStruct((N_TOK,), jnp.int32),
        scratch_types=[
            pltpu.VMEM((D_LAT,), jnp.float32),
            pltpu.VMEM((D_LAT,), jnp.float32),
            pltpu.VMEM((ROWS_PER_W, NCHUNK), jnp.float32),
            pltpu.VMEM((CAND_N,), jnp.int32),
            pltpu.VMEM((ROWS_PER_W,), jnp.int32),
            pltpu.SemaphoreType.DMA,
            pltpu.SemaphoreType.DMA,
            pltpu.SemaphoreType.DMA,
        ],
    )
    def k(pre_hbm, m_hbm, thr_hbm, row0, row1, mslab, cand, thrbuf,
          sem0, sem1, semm):
        wid = lax.axis_index("s") * 2 + lax.axis_index("c")
        base = wid * ROWS_PER_W
        pltpu.async_copy(m_hbm.at[pl.ds(base, ROWS_PER_W)], mslab, semm).wait()
        pltpu.async_copy(pre_hbm.at[base], row0, sem0)

        def do_row(r, cur, cur_sem, nxt, nxt_sem):
            pltpu.make_async_copy(pre_hbm.at[base + r], cur, cur_sem).wait()

            @pl.when(r + 1 < ROWS_PER_W)
            def _():
                pltpu.async_copy(pre_hbm.at[base + r + 1], nxt, nxt_sem)

            def count_m(mid):
                midv = jnp.full((16,), mid, jnp.int32)

                def body(j, acc):
                    bits = plsc.bitcast(mslab[r, pl.ds(j * 16, 16)], jnp.int32)
                    return acc + (bits >= midv).astype(jnp.int32)

                accv = lax.fori_loop(0, NCHUNK // 16, body,
                                     jnp.zeros((16,), jnp.int32))
                return jnp.sum(accv)

            t0 = _bisect(count_m, K, 1, INF_BITS + 8)

            @pl.loop(0, LANE_SEG)
            def _(j):
                cand[pl.ds(j * 16, 16)] = jnp.zeros((16,), jnp.int32)

            t0v = jnp.full((16,), t0, jnp.int32)
            lanebase = lax.iota(jnp.int32, 16) * LANE_SEG

            def cbody(j, cnt16):
                bits = plsc.bitcast(cur[pl.ds(j * 16, 16)], jnp.int32)
                mask = bits >= t0v
                dest = lanebase + jnp.minimum(cnt16, LANE_SEG - 1)
                plsc.store_scatter(cand, [dest], bits, mask=mask)
                return cnt16 + mask.astype(jnp.int32)

            cnt16 = lax.fori_loop(0, D_LAT // 16, cbody,
                                  jnp.zeros((16,), jnp.int32))
            n_cand = jnp.sum(cnt16)

            need = jnp.minimum(K, n_cand)

            def count_c(mid):
                midv = jnp.full((16,), mid, jnp.int32)

                def body(j, acc):
                    return acc + (cand[pl.ds(j * 16, 16)] >= midv).astype(
                        jnp.int32)

                accv = lax.fori_loop(0, LANE_SEG, body,
                                     jnp.zeros((16,), jnp.int32))
                return jnp.sum(accv)

            thr = _bisect(count_c, need, t0, INF_BITS + 8)
            thr = jnp.where(n_cand == 0, INF_BITS, thr)
            lane = lax.iota(jnp.int32, 16)
            plsc.store_scatter(thrbuf, [jnp.full((16,), r, jnp.int32)],
                               jnp.full((16,), thr, jnp.int32),
                               mask=lane == 0)

        @pl.loop(0, ROWS_PER_W, step=2)
        def _(r):
            do_row(r, row0, sem0, row1, sem1)
            do_row(r + 1, row1, sem1, row0, sem0)

        pltpu.sync_copy(thrbuf, thr_hbm.at[pl.ds(base, ROWS_PER_W)])

    thr_bits = k(pre, M)
    return lax.bitcast_convert_type(thr_bits, jnp.float32).reshape(N_TOK, 1)


def _dec_body(pre_ref, thr_ref, w_ref, b_ref, z_ref, o_ref):
    l = pl.program_id(1)
    pre = pre_ref[...]
    thr = thr_ref[...]
    z = jnp.where((pre >= thr) & (pre > 0.0), pre, 0.0)
    z_ref[...] = z
    acc = jax.lax.dot_general(
        z.astype(jnp.bfloat16), w_ref[...],
        dimension_numbers=(((1,), (1,)), ((), ())),
        preferred_element_type=jnp.float32,
    )

    @pl.when(l == 0)
    def _():
        o_ref[...] = acc + b_ref[...]

    @pl.when(l != 0)
    def _():
        o_ref[...] += acc


def _decoder(pre, thr, W_dec, b_dec):
    b2 = b_dec.reshape(1, D_IN)
    grid = (N_TOK // DEC_BT, D_LAT // DEC_BL)
    return pl.pallas_call(
        _dec_body,
        grid=grid,
        in_specs=[
            pl.BlockSpec((DEC_BT, DEC_BL), lambda t, l: (t, l)),
            pl.BlockSpec((DEC_BT, 1), lambda t, l: (t, 0)),
            pl.BlockSpec((D_IN, DEC_BL), lambda t, l: (0, l)),
            pl.BlockSpec((1, D_IN), lambda t, l: (0, 0)),
        ],
        out_specs=[
            pl.BlockSpec((DEC_BT, DEC_BL), lambda t, l: (t, l)),
            pl.BlockSpec((DEC_BT, D_IN), lambda t, l: (t, 0)),
        ],
        out_shape=[
            jax.ShapeDtypeStruct((N_TOK, D_LAT), jnp.float32),
            jax.ShapeDtypeStruct((N_TOK, D_IN), jnp.float32),
        ],
    )(pre, thr, W_dec.astype(jnp.bfloat16), b2)


def kernel(x, W_enc, b_enc, W_dec, b_dec):
    pre, M3 = _encoder(x, W_enc, b_enc)
    M = M3.transpose(1, 0, 2).reshape(N_TOK, NCHUNK)
    thr = _sc_thresholds(pre, M)
    z, x_hat = _decoder(pre, thr, W_dec, b_dec)
    return (x_hat, z)


# speedup vs baseline: 9.5640x; 1.2280x over previous
import dataclasses
import functools

import jax
import jax.numpy as jnp
from jax import lax
from jax.experimental import pallas as pl
from jax.experimental.pallas import tpu as pltpu
from jax.experimental.pallas import tpu_sc as plsc

N_TOK = 4096
D_IN = 2048
D_LAT = 16384
K = 64

ENC_BT = 256
ENC_BL = 2048
DEC_BT = 512
DEC_BL = 1024
CHUNK = 128
NCHUNK = D_LAT // CHUNK

NW = 32
ROWS_PER_W = N_TOK // NW
LANE_SEG = 32
CAND_N = 16 * LANE_SEG
INF_BITS = 0x7F800000


def _enc_body(x_ref, w_ref, b_ref, o_ref, m_ref):
    acc = jax.lax.dot_general(
        x_ref[...], w_ref[...],
        dimension_numbers=(((1,), (1,)), ((), ())),
        preferred_element_type=jnp.float32,
    )
    pre = jnp.maximum(acc + b_ref[...], 0.0)
    o_ref[...] = pre
    m = jnp.max(pre.reshape(ENC_BT, ENC_BL // CHUNK, CHUNK), axis=2)
    m_ref[...] = m.reshape(1, ENC_BT, ENC_BL // CHUNK)


def _encoder(x, W_enc, b_enc):
    b2 = b_enc.reshape(1, D_LAT)
    grid = (D_LAT // ENC_BL, N_TOK // ENC_BT)
    return pl.pallas_call(
        _enc_body,
        grid=grid,
        in_specs=[
            pl.BlockSpec((ENC_BT, D_IN), lambda l, t: (t, 0)),
            pl.BlockSpec((ENC_BL, D_IN), lambda l, t: (l, 0)),
            pl.BlockSpec((1, ENC_BL), lambda l, t: (0, l)),
        ],
        out_specs=[
            pl.BlockSpec((ENC_BT, ENC_BL), lambda l, t: (t, l)),
            pl.BlockSpec((1, ENC_BT, ENC_BL // CHUNK), lambda l, t: (l, t, 0)),
        ],
        out_shape=[
            jax.ShapeDtypeStruct((N_TOK, D_LAT), jnp.float32),
            jax.ShapeDtypeStruct((D_LAT // ENC_BL, N_TOK, ENC_BL // CHUNK),
                                 jnp.float32),
        ],
    )(x.astype(jnp.bfloat16), W_enc.astype(jnp.bfloat16), b2)


def _count_ge_f32(ref, nv, mid):
    midv = jnp.full((16,), mid, jnp.int32)

    def body(j, acc):
        bits = plsc.bitcast(ref[pl.ds(j * 16, 16)], jnp.int32)
        return acc + jnp.sum((bits >= midv).astype(jnp.int32))

    return lax.fori_loop(0, nv, body, 0)


def _bisect(count_fn, need, lo, hi):
    def cond(c):
        lo, hi = c
        return hi - lo > 1

    def it(c):
        lo, hi = c
        mid = lo + (hi - lo) // 2
        good = count_fn(mid) >= need
        return (jnp.where(good, mid, lo), jnp.where(good, hi, mid))

    lo, hi = lax.while_loop(cond, it, (lo, hi))
    return lo


def _sc_thresholds(pre, M):
    mesh = plsc.VectorSubcoreMesh(core_axis_name="c", subcore_axis_name="s")
    cp = pltpu.CompilerParams()
    if "needs_layout_passes" in pltpu.CompilerParams.__dataclass_fields__:
        cp = dataclasses.replace(cp, needs_layout_passes=False)

    @functools.partial(
        pl.kernel,
        mesh=mesh,
        compiler_params=cp,
        out_type=jax.ShapeDtypeStruct((N_TOK,), jnp.int32),
        scratch_types=[
            pltpu.VMEM((D_LAT,), jnp.float32),
            pltpu.VMEM((D_LAT,), jnp.float32),
            pltpu.VMEM((ROWS_PER_W, NCHUNK), jnp.float32),
            pltpu.VMEM((CAND_N,), jnp.int32),
            pltpu.VMEM((ROWS_PER_W,), jnp.int32),
            pltpu.SemaphoreType.DMA,
            pltpu.SemaphoreType.DMA,
            pltpu.SemaphoreType.DMA,
        ],
    )
    def k(pre_hbm, m_hbm, thr_hbm, row0, row1, mslab, cand, thrbuf,
          sem0, sem1, semm):
        wid = lax.axis_index("s") * 2 + lax.axis_index("c")
        base = wid * ROWS_PER_W
        pltpu.async_copy(m_hbm.at[pl.ds(base, ROWS_PER_W)], mslab, semm).wait()
        pltpu.async_copy(pre_hbm.at[base], row0, sem0)

        def do_row(r, cur, cur_sem, nxt, nxt_sem):
            pltpu.make_async_copy(pre_hbm.at[base + r], cur, cur_sem).wait()

            @pl.when(r + 1 < ROWS_PER_W)
            def _():
                pltpu.async_copy(pre_hbm.at[base + r + 1], nxt, nxt_sem)

            def count_m(mid):
                midv = jnp.full((16,), mid, jnp.int32)

                def body(j, acc):
                    bits = plsc.bitcast(mslab[r, pl.ds(j * 16, 16)], jnp.int32)
                    return acc + (bits >= midv).astype(jnp.int32)

                accv = lax.fori_loop(0, NCHUNK // 16, body,
                                     jnp.zeros((16,), jnp.int32), unroll=8)
                return jnp.sum(accv)

            t0 = _bisect(count_m, K, 1, INF_BITS + 8)

            @pl.loop(0, LANE_SEG)
            def _(j):
                cand[pl.ds(j * 16, 16)] = jnp.zeros((16,), jnp.int32)

            t0v = jnp.full((16,), t0, jnp.int32)
            lanebase = lax.iota(jnp.int32, 16) * LANE_SEG

            def cbody(j, cnt16):
                bits = plsc.bitcast(cur[pl.ds(j * 16, 16)], jnp.int32)
                mask = bits >= t0v
                dest = lanebase + jnp.minimum(cnt16, LANE_SEG - 1)
                plsc.store_scatter(cand, [dest], bits, mask=mask)
                return cnt16 + mask.astype(jnp.int32)

            cnt16 = lax.fori_loop(0, D_LAT // 16, cbody,
                                  jnp.zeros((16,), jnp.int32), unroll=8)
            n_cand = jnp.sum(cnt16)

            need = jnp.minimum(K, n_cand)

            def count_c(mid):
                midv = jnp.full((16,), mid, jnp.int32)

                def body(j, acc):
                    return acc + (cand[pl.ds(j * 16, 16)] >= midv).astype(
                        jnp.int32)

                accv = lax.fori_loop(0, LANE_SEG, body,
                                     jnp.zeros((16,), jnp.int32), unroll=8)
                return jnp.sum(accv)

            thr = _bisect(count_c, need, t0, INF_BITS + 8)
            thr = jnp.where(n_cand == 0, INF_BITS, thr)
            lane = lax.iota(jnp.int32, 16)
            plsc.store_scatter(thrbuf, [jnp.full((16,), r, jnp.int32)],
                               jnp.full((16,), thr, jnp.int32),
                               mask=lane == 0)

        @pl.loop(0, ROWS_PER_W, step=2)
        def _(r):
            do_row(r, row0, sem0, row1, sem1)
            do_row(r + 1, row1, sem1, row0, sem0)

        pltpu.sync_copy(thrbuf, thr_hbm.at[pl.ds(base, ROWS_PER_W)])

    thr_bits = k(pre, M)
    return lax.bitcast_convert_type(thr_bits, jnp.float32).reshape(N_TOK, 1)


def _dec_body(pre_ref, thr_ref, w_ref, b_ref, z_ref, o_ref):
    l = pl.program_id(1)
    pre = pre_ref[...]
    thr = thr_ref[...]
    z = jnp.where((pre >= thr) & (pre > 0.0), pre, 0.0)
    z_ref[...] = z
    acc = jax.lax.dot_general(
        z.astype(jnp.bfloat16), w_ref[...],
        dimension_numbers=(((1,), (1,)), ((), ())),
        preferred_element_type=jnp.float32,
    )

    @pl.when(l == 0)
    def _():
        o_ref[...] = acc + b_ref[...]

    @pl.when(l != 0)
    def _():
        o_ref[...] += acc


def _decoder(pre, thr, W_dec, b_dec):
    b2 = b_dec.reshape(1, D_IN)
    grid = (N_TOK // DEC_BT, D_LAT // DEC_BL)
    return pl.pallas_call(
        _dec_body,
        grid=grid,
        in_specs=[
            pl.BlockSpec((DEC_BT, DEC_BL), lambda t, l: (t, l)),
            pl.BlockSpec((DEC_BT, 1), lambda t, l: (t, 0)),
            pl.BlockSpec((D_IN, DEC_BL), lambda t, l: (0, l)),
            pl.BlockSpec((1, D_IN), lambda t, l: (0, 0)),
        ],
        out_specs=[
            pl.BlockSpec((DEC_BT, DEC_BL), lambda t, l: (t, l)),
            pl.BlockSpec((DEC_BT, D_IN), lambda t, l: (t, 0)),
        ],
        out_shape=[
            jax.ShapeDtypeStruct((N_TOK, D_LAT), jnp.float32),
            jax.ShapeDtypeStruct((N_TOK, D_IN), jnp.float32),
        ],
    )(pre, thr, W_dec.astype(jnp.bfloat16), b2)


def kernel(x, W_enc, b_enc, W_dec, b_dec):
    pre, M3 = _encoder(x, W_enc, b_enc)
    M = M3.transpose(1, 0, 2).reshape(N_TOK, NCHUNK)
    thr = _sc_thresholds(pre, M)
    z, x_hat = _decoder(pre, thr, W_dec, b_dec)
    return (x_hat, z)


# speedup vs baseline: 11.2068x; 1.1718x over previous
import dataclasses
import functools

import jax
import jax.numpy as jnp
from jax import lax
from jax.experimental import pallas as pl
from jax.experimental.pallas import tpu as pltpu
from jax.experimental.pallas import tpu_sc as plsc

N_TOK = 4096
D_IN = 2048
D_LAT = 16384
K = 64

ENC_BT = 256
ENC_BL = 2048
DEC_BT = 512
DEC_BL = 1024
CHUNK = 128
NCHUNK = D_LAT // CHUNK

NW = 32
ROWS_PER_W = N_TOK // NW
LANE_SEG = 32
CAND_N = 16 * LANE_SEG
INF_BITS = 0x7F800000


def _enc_body(x_ref, w_ref, b_ref, o_ref, m_ref):
    acc = jax.lax.dot_general(
        x_ref[...], w_ref[...],
        dimension_numbers=(((1,), (1,)), ((), ())),
        preferred_element_type=jnp.float32,
    )
    pre = jnp.maximum(acc + b_ref[...], 0.0)
    o_ref[...] = pre
    m = jnp.max(pre.reshape(ENC_BT, ENC_BL // CHUNK, CHUNK), axis=2)
    m_ref[...] = m.reshape(1, ENC_BT, ENC_BL // CHUNK)


def _encoder(xb, wb, b2, n_tok):
    grid = (D_LAT // ENC_BL, n_tok // ENC_BT)
    return pl.pallas_call(
        _enc_body,
        grid=grid,
        in_specs=[
            pl.BlockSpec((ENC_BT, D_IN), lambda l, t: (t, 0)),
            pl.BlockSpec((ENC_BL, D_IN), lambda l, t: (l, 0)),
            pl.BlockSpec((1, ENC_BL), lambda l, t: (0, l)),
        ],
        out_specs=[
            pl.BlockSpec((ENC_BT, ENC_BL), lambda l, t: (t, l)),
            pl.BlockSpec((1, ENC_BT, ENC_BL // CHUNK), lambda l, t: (l, t, 0)),
        ],
        out_shape=[
            jax.ShapeDtypeStruct((n_tok, D_LAT), jnp.float32),
            jax.ShapeDtypeStruct((D_LAT // ENC_BL, n_tok, ENC_BL // CHUNK),
                                 jnp.float32),
        ],
    )(xb, wb, b2)


def _count_ge_f32(ref, nv, mid):
    midv = jnp.full((16,), mid, jnp.int32)

    def body(j, acc):
        bits = plsc.bitcast(ref[pl.ds(j * 16, 16)], jnp.int32)
        return acc + jnp.sum((bits >= midv).astype(jnp.int32))

    return lax.fori_loop(0, nv, body, 0)


def _bisect(count_fn, need, lo, hi):
    def cond(c):
        lo, hi = c
        return hi - lo > 1

    def it(c):
        lo, hi = c
        mid = lo + (hi - lo) // 2
        good = count_fn(mid) >= need
        return (jnp.where(good, mid, lo), jnp.where(good, hi, mid))

    lo, hi = lax.while_loop(cond, it, (lo, hi))
    return lo


def _sc_thresholds(pre, M, n_tok):
    rows_w = n_tok // NW
    mesh = plsc.VectorSubcoreMesh(core_axis_name="c", subcore_axis_name="s")
    cp = pltpu.CompilerParams()
    if "needs_layout_passes" in pltpu.CompilerParams.__dataclass_fields__:
        cp = dataclasses.replace(cp, needs_layout_passes=False)

    @functools.partial(
        pl.kernel,
        mesh=mesh,
        compiler_params=cp,
        out_type=jax.ShapeDtypeStruct((n_tok,), jnp.int32),
        scratch_types=[
            pltpu.VMEM((D_LAT,), jnp.float32),
            pltpu.VMEM((D_LAT,), jnp.float32),
            pltpu.VMEM((rows_w, NCHUNK), jnp.float32),
            pltpu.VMEM((CAND_N,), jnp.int32),
            pltpu.VMEM((rows_w,), jnp.int32),
            pltpu.SemaphoreType.DMA,
            pltpu.SemaphoreType.DMA,
            pltpu.SemaphoreType.DMA,
        ],
    )
    def k(pre_hbm, m_hbm, thr_hbm, row0, row1, mslab, cand, thrbuf,
          sem0, sem1, semm):
        wid = lax.axis_index("s") * 2 + lax.axis_index("c")
        base = wid * rows_w
        pltpu.async_copy(m_hbm.at[pl.ds(base, rows_w)], mslab, semm).wait()
        pltpu.async_copy(pre_hbm.at[base], row0, sem0)

        def do_row(r, cur, cur_sem, nxt, nxt_sem):
            pltpu.make_async_copy(pre_hbm.at[base + r], cur, cur_sem).wait()

            @pl.when(r + 1 < rows_w)
            def _():
                pltpu.async_copy(pre_hbm.at[base + r + 1], nxt, nxt_sem)

            def count_m(mid):
                midv = jnp.full((16,), mid, jnp.int32)

                def body(j, acc):
                    bits = plsc.bitcast(mslab[r, pl.ds(j * 16, 16)], jnp.int32)
                    return acc + (bits >= midv).astype(jnp.int32)

                accv = lax.fori_loop(0, NCHUNK // 16, body,
                                     jnp.zeros((16,), jnp.int32), unroll=8)
                return jnp.sum(accv)

            t0 = _bisect(count_m, K, 1, INF_BITS + 8)

            @pl.loop(0, LANE_SEG)
            def _(j):
                cand[pl.ds(j * 16, 16)] = jnp.zeros((16,), jnp.int32)

            t0v = jnp.full((16,), t0, jnp.int32)
            lanebase = lax.iota(jnp.int32, 16) * LANE_SEG

            def cbody(j, cnt16):
                bits = plsc.bitcast(cur[pl.ds(j * 16, 16)], jnp.int32)
                mask = bits >= t0v
                dest = lanebase + jnp.minimum(cnt16, LANE_SEG - 1)
                plsc.store_scatter(cand, [dest], bits, mask=mask)
                return cnt16 + mask.astype(jnp.int32)

            cnt16 = lax.fori_loop(0, D_LAT // 16, cbody,
                                  jnp.zeros((16,), jnp.int32), unroll=8)
            n_cand = jnp.sum(cnt16)

            need = jnp.minimum(K, n_cand)

            def count_c(mid):
                midv = jnp.full((16,), mid, jnp.int32)

                def body(j, acc):
                    return acc + (cand[pl.ds(j * 16, 16)] >= midv).astype(
                        jnp.int32)

                accv = lax.fori_loop(0, LANE_SEG, body,
                                     jnp.zeros((16,), jnp.int32), unroll=8)
                return jnp.sum(accv)

            thr = _bisect(count_c, need, t0, INF_BITS + 8)
            thr = jnp.where(n_cand == 0, INF_BITS, thr)
            lane = lax.iota(jnp.int32, 16)
            plsc.store_scatter(thrbuf, [jnp.full((16,), r, jnp.int32)],
                               jnp.full((16,), thr, jnp.int32),
                               mask=lane == 0)

        @pl.loop(0, rows_w, step=2)
        def _(r):
            do_row(r, row0, sem0, row1, sem1)
            do_row(r + 1, row1, sem1, row0, sem0)

        pltpu.sync_copy(thrbuf, thr_hbm.at[pl.ds(base, rows_w)])

    thr_bits = k(pre, M)
    return lax.bitcast_convert_type(thr_bits, jnp.float32).reshape(n_tok, 1)


def _dec_body(pre_ref, thr_ref, w_ref, b_ref, z_ref, o_ref):
    l = pl.program_id(1)
    pre = pre_ref[...]
    thr = thr_ref[...]
    z = jnp.where((pre >= thr) & (pre > 0.0), pre, 0.0)
    z_ref[...] = z
    acc = jax.lax.dot_general(
        z.astype(jnp.bfloat16), w_ref[...],
        dimension_numbers=(((1,), (1,)), ((), ())),
        preferred_element_type=jnp.float32,
    )

    @pl.when(l == 0)
    def _():
        o_ref[...] = acc + b_ref[...]

    @pl.when(l != 0)
    def _():
        o_ref[...] += acc


def _decoder(pre, thr, wdb, b2, n_tok):
    grid = (n_tok // DEC_BT, D_LAT // DEC_BL)
    return pl.pallas_call(
        _dec_body,
        grid=grid,
        in_specs=[
            pl.BlockSpec((DEC_BT, DEC_BL), lambda t, l: (t, l)),
            pl.BlockSpec((DEC_BT, 1), lambda t, l: (t, 0)),
            pl.BlockSpec((D_IN, DEC_BL), lambda t, l: (0, l)),
            pl.BlockSpec((1, D_IN), lambda t, l: (0, 0)),
        ],
        out_specs=[
            pl.BlockSpec((DEC_BT, DEC_BL), lambda t, l: (t, l)),
            pl.BlockSpec((DEC_BT, D_IN), lambda t, l: (t, 0)),
        ],
        out_shape=[
            jax.ShapeDtypeStruct((n_tok, D_LAT), jnp.float32),
            jax.ShapeDtypeStruct((n_tok, D_IN), jnp.float32),
        ],
    )(pre, thr, wdb, b2)


N_CHUNKS = 4
CHUNK_TOK = N_TOK // N_CHUNKS


def kernel(x, W_enc, b_enc, W_dec, b_dec):
    xb = x.astype(jnp.bfloat16)
    wb = W_enc.astype(jnp.bfloat16)
    wdb = W_dec.astype(jnp.bfloat16)
    b2e = b_enc.reshape(1, D_LAT)
    b2d = b_dec.reshape(1, D_IN)
    z_parts, xh_parts = [], []
    for c in range(N_CHUNKS):
        xc = lax.slice_in_dim(xb, c * CHUNK_TOK, (c + 1) * CHUNK_TOK, axis=0)
        pre, M3 = _encoder(xc, wb, b2e, CHUNK_TOK)
        M = M3.transpose(1, 0, 2).reshape(CHUNK_TOK, NCHUNK)
        thr = _sc_thresholds(pre, M, CHUNK_TOK)
        z_c, xh_c = _decoder(pre, thr, wdb, b2d, CHUNK_TOK)
        z_parts.append(z_c)
        xh_parts.append(xh_c)
    z = jnp.concatenate(z_parts, axis=0)
    x_hat = jnp.concatenate(xh_parts, axis=0)
    return (x_hat, z)
